# Initial kernel scaffold; baseline (speedup 1.0000x reference)
import jax
import jax.numpy as jnp
from jax.experimental import pallas as pl


def kernel(input, codebook):
    raise NotImplementedError("write your pallas kernel here")


# speedup vs baseline: 1.0660x; 1.0660x over previous
import jax
import jax.numpy as jnp
from jax.experimental import pallas as pl
from jax.experimental.pallas import tpu as pltpu
from jax.experimental.pallas import tpu_sc as plsc

CB = 8192
ED = 256
BM = 256
CT = 1024
BIGI = 2 ** 30


STRIP_ENDS = (2736, 5472)


def _argmin_body(zsq_ref, z_ref, cbm2_ref, esq_ref, idx_ref):
    z = z_ref[...]
    zsq = zsq_ref[0]
    best = jnp.full((BM, 1), jnp.inf, jnp.float32)
    bidx = jnp.full((BM, 1), BIGI, jnp.int32)
    for ct in range(CB // CT):
        lo = ct * CT
        cb_t = cbm2_ref[lo:lo + CT, :]
        d = jax.lax.dot_general(
            z, cb_t,
            dimension_numbers=(((1,), (1,)), ((), ())),
            preferred_element_type=jnp.float32)
        t = zsq + d
        dist = t + esq_ref[:, lo:lo + CT]
        iota = jax.lax.broadcasted_iota(jnp.int32, (BM, CT), 1) + lo
        cuts = [b for b in STRIP_ENDS if lo < b < lo + CT]
        seg_edges = [lo] + cuts + [lo + CT]
        for s0, s1 in zip(seg_edges[:-1], seg_edges[1:]):
            if s1 - s0 == CT:
                seg = dist
                smin = jnp.min(seg, axis=1, keepdims=True)
                sidx = jnp.min(jnp.where(seg == smin, iota, BIGI),
                               axis=1, keepdims=True)
            else:
                mask = (iota >= s0) & (iota < s1)
                seg = jnp.where(mask, dist, jnp.inf)
                smin = jnp.min(seg, axis=1, keepdims=True)
                sidx = jnp.min(jnp.where(mask & (dist == smin), iota, BIGI),
                               axis=1, keepdims=True)
            upd = smin < best
            best = jnp.where(upd, smin, best)
            bidx = jnp.where(upd, sidx, bidx)
            if s1 in STRIP_ENDS:
                best = best.astype(jnp.bfloat16).astype(jnp.float32)
    idx_ref[0] = bidx


def _tc_argmin(z_flat, zsq, cbm2, esq):
    m = z_flat.shape[0]
    nblk = m // BM
    return pl.pallas_call(
        _argmin_body,
        grid=(nblk,),
        in_specs=[
            pl.BlockSpec((1, BM, 1), lambda i: (i, 0, 0)),
            pl.BlockSpec((BM, ED), lambda i: (i, 0)),
            pl.BlockSpec((CB, ED), lambda i: (0, 0)),
            pl.BlockSpec((1, CB), lambda i: (0, 0)),
        ],
        out_specs=pl.BlockSpec((1, BM, 1), lambda i: (i, 0, 0)),
        out_shape=jax.ShapeDtypeStruct((nblk, BM, 1), jnp.int32),
    )(zsq.reshape(nblk, BM, 1), z_flat, cbm2, esq.reshape(1, CB))


def _sc_gather(codebook, idx_row, n):
    win = 128
    mesh = plsc.VectorSubcoreMesh(core_axis_name="core",
                                  subcore_axis_name="subcore")

    @pl.kernel(out_type=jax.ShapeDtypeStruct((n, ED), codebook.dtype),
               mesh=mesh)
    def k(cb_hbm, i_hbm, o_hbm):
        def body(i_vmem, o_vmem):
            pltpu.sync_copy(cb_hbm.at[i_vmem.at[0]], o_vmem)

        pltpu.emit_pipeline(
            body,
            grid=(n // win,),
            in_specs=[pl.BlockSpec((1, win), index_map=lambda i: (0, i))],
            out_specs=[pl.BlockSpec((win, ED), index_map=lambda i: (i, 0))],
            core_axis_name=("core", "subcore"),
            dimension_semantics=(pltpu.PARALLEL,),
        )(i_hbm, o_hbm)

    return k(codebook, idx_row)


def kernel(input, codebook):
    b = input.shape[0]
    d = input.shape[1]
    spatial = input.shape[2:]
    z = input.reshape(b, d, -1)
    z = jnp.transpose(z, (0, 2, 1))
    z_flat = z.reshape(-1, d)
    z_flat = jax.lax.optimization_barrier(z_flat)
    zsq = jnp.sum(z_flat ** 2, axis=-1, keepdims=True)
    esq = jnp.sum(codebook ** 2, axis=-1)
    z16 = z_flat.astype(jnp.bfloat16)
    cbm2 = (codebook * (-2.0)).astype(jnp.bfloat16)

    m = z_flat.shape[0]
    idx3 = _tc_argmin(z16, zsq, cbm2, esq)
    idx_flat = idx3.reshape(m)
    quant = _sc_gather(codebook, idx_flat.reshape(1, m), m)

    quant = quant.reshape(b, -1, d)
    quant = jnp.transpose(quant, (0, 2, 1))
    output = quant.reshape(input.shape)
    indices = idx_flat.reshape((b,) + spatial).astype(jnp.int64)
    return output, indices
